# baseline (speedup 1.0000x reference)
import functools

import jax
import jax.numpy as jnp
from jax import lax
from jax.experimental import pallas as pl
from jax.experimental.pallas import tpu as pltpu
from jax.experimental.pallas import tpu_sc as plsc

N = 320000
D = 128
NUM_SEGMENTS = 10000

NUM_WORKERS = 32
CHUNK = 128
NBUF = 3
SEG_PAD = 10112
SEG_PER_TILE = SEG_PAD // 16

U_ROWS = N // D
A_BLOCK = 320
A_GRID = 4
STAGE_UROWS = A_BLOCK * A_GRID
STAGE0_CHUNKS = STAGE_UROWS
STAGE1_CHUNKS = U_ROWS - STAGE_UROWS


def _weights_body(x_ref, w_ref, u_ref):
    m = x_ref[...] * w_ref[...][:, None, :]
    u_ref[...] = jnp.sum(m, axis=-1)
    u_ref[...] = jnp.exp(u_ref[...])


def _weights(x3, att_w, half):
    return pl.pallas_call(
        _weights_body,
        grid=(A_GRID,),
        in_specs=[
            pl.BlockSpec((A_BLOCK, D, D), lambda i: (i + A_GRID * half, 0, 0)),
            pl.BlockSpec((1, D), lambda i: (0, 0)),
        ],
        out_specs=pl.BlockSpec((A_BLOCK, D), lambda i: (i, 0)),
        out_shape=jax.ShapeDtypeStruct((STAGE_UROWS, D), jnp.float32),
    )(x3, att_w)


def _sc_body(co, total_chunks, x_hbm, u_hbm, b_hbm, out_hbm, acc,
             xb0, xb1, xb2, ubuf, ibuf, s0, s1, s2, o0, o1, o2):
    base_chunks = total_chunks // NUM_WORKERS
    extra_workers = total_chunks - base_chunks * NUM_WORKERS
    xbufs = [xb0, xb1, xb2]
    sems = [s0, s1, s2]
    osems = [o0, o1, o2]
    cid = lax.axis_index("c")
    sid = lax.axis_index("s")
    wid = cid * 16 + sid
    start = base_chunks * wid + jnp.minimum(wid, extra_workers)
    has_extra = wid < extra_workers
    nloc = base_chunks + jnp.where(has_extra, 1, 0)

    def _copies(c, b):
        base = pl.multiple_of(c * CHUNK, CHUNK)
        gbase = pl.multiple_of((c + co) * CHUNK, CHUNK)
        return (
            pltpu.make_async_copy(x_hbm.at[pl.ds(gbase, CHUNK)], xbufs[b], sems[b]),
            pltpu.make_async_copy(u_hbm.at[pl.ds(base, CHUNK)], ubuf.at[b], sems[b]),
            pltpu.make_async_copy(b_hbm.at[pl.ds(gbase, CHUNK)], ibuf.at[b], sems[b]),
        )

    def _prime(c, b):
        for d in _copies(c, b):
            d.start()

    def _wait_in(c, b):
        for d in _copies(c, b):
            d.wait()

    def _wait_scatter(b):
        pltpu.make_async_copy(xbufs[b], acc.at[ibuf.at[b]], osems[b]).wait()

    _prime(start, 0)
    _prime(start + 1, 1)

    def zrow(i, _):
        for jj in range(D // 16):
            xb2[i, pl.ds(jj * 16, 16)] = jnp.zeros((16,), jnp.float32)
        return 0
    lax.fori_loop(0, CHUNK, zrow, 0)
    stripe = sid * SEG_PER_TILE
    for k in range(SEG_PER_TILE // CHUNK):
        pltpu.sync_copy(xb2, acc.at[pl.ds(stripe + k * CHUNK, CHUNK)])
    rem = SEG_PER_TILE % CHUNK
    if rem:
        pltpu.sync_copy(
            xb2.at[pl.ds(0, rem)],
            acc.at[pl.ds(stripe + (SEG_PER_TILE // CHUNK) * CHUNK, rem)])

    _prime(start + 2, 2)
    plsc.subcore_barrier()

    def _scale(xb, b):
        def grp(t, _):
            uvec = ubuf[b, pl.ds(t * 16, 16)]
            for i in range(16):
                val = uvec[i]
                for jj in range(D // 16):
                    sl = pl.ds(jj * 16, 16)
                    xb[t * 16 + i, sl] = xb[t * 16 + i, sl] * val
            return 0
        lax.fori_loop(0, CHUNK // 16, grp, 0)

    def _step(j, b):
        c = start + j
        _wait_in(c, b)
        _scale(xbufs[b], b)
        pltpu.async_copy(xbufs[b], acc.at[ibuf.at[b]], osems[b], add=True)
        bq = (b + 2) % NBUF

        @pl.when(j >= 1)
        def _():
            _wait_scatter(bq)

        @pl.when((j >= 1) & (j + 2 < nloc))
        def _():
            _prime(c + 2, bq)

    main_iters = base_chunks // NBUF

    def outer(k, _):
        for b in range(NBUF):
            _step(k * NBUF + b, b)
        return 0
    lax.fori_loop(0, main_iters, outer, 0)

    for jj in range((base_chunks % NBUF) + 1):
        j = main_iters * NBUF + jj

        @pl.when(j < nloc)
        def _():
            _step(j, j % NBUF)

    @pl.when(has_extra)
    def _():
        _wait_scatter(base_chunks % NBUF)

    @pl.when(jnp.logical_not(has_extra))
    def _():
        _wait_scatter((base_chunks - 1) % NBUF)

    plsc.subcore_barrier()
    pltpu.sync_copy(acc.at[pl.ds(stripe, SEG_PER_TILE)],
                    out_hbm.at[cid, pl.ds(stripe, SEG_PER_TILE)])


def _sc_scatter(x, u_flat, batch32, co, total_chunks):
    mesh = plsc.VectorSubcoreMesh(core_axis_name="c", subcore_axis_name="s")
    f = pl.kernel(
        functools.partial(_sc_body, co, total_chunks),
        out_type=jax.ShapeDtypeStruct((2, SEG_PAD, D), jnp.float32),
        mesh=mesh,
        scratch_types=[
            pltpu.VMEM_SHARED((SEG_PAD, D), jnp.float32),
            pltpu.VMEM((CHUNK, D), jnp.float32),
            pltpu.VMEM((CHUNK, D), jnp.float32),
            pltpu.VMEM((CHUNK, D), jnp.float32),
            pltpu.VMEM((NBUF, CHUNK), jnp.float32),
            pltpu.VMEM((NBUF, CHUNK), jnp.int32),
            pltpu.SemaphoreType.DMA,
            pltpu.SemaphoreType.DMA,
            pltpu.SemaphoreType.DMA,
            pltpu.SemaphoreType.DMA,
            pltpu.SemaphoreType.DMA,
            pltpu.SemaphoreType.DMA,
        ],
    )
    return f(x, u_flat, batch32)


def _combine_body(pa_ref, pb_ref, ua_ref, ub_ref, o_ref):
    z = jnp.sum(ua_ref[...]) + jnp.sum(ub_ref[:STAGE1_CHUNKS])
    o_ref[...] = (pa_ref[0, :NUM_SEGMENTS] + pa_ref[1, :NUM_SEGMENTS]
                  + pb_ref[0, :NUM_SEGMENTS] + pb_ref[1, :NUM_SEGMENTS]) * (1.0 / z)


def _combine(pa, pb, ua, ub):
    return pl.pallas_call(
        _combine_body,
        out_shape=jax.ShapeDtypeStruct((NUM_SEGMENTS, D), jnp.float32),
    )(pa, pb, ua, ub)


@jax.jit
def kernel(x, batch, att_w, att_b):
    del att_b
    x3 = x.reshape(N // D, D, D)
    batch32 = batch.astype(jnp.int32)
    ua = _weights(x3, att_w, 0)
    pa = _sc_scatter(x, ua.reshape(-1), batch32, 0, STAGE0_CHUNKS)
    ub = _weights(x3, att_w, 1)
    pb = _sc_scatter(x, ub.reshape(-1), batch32, STAGE0_CHUNKS, STAGE1_CHUNKS)
    return _combine(pa, pb, ua, ub)
